# baseline (speedup 1.0000x reference)
import jax
import jax.numpy as jnp
from jax.experimental import pallas as pl
from jax.experimental.pallas import tpu as pltpu

C = 4
H = 16
W = 16
N_STEPS = 8
R = C * H
RA = R + 8
RS = 208
NB = 512


def _band_weights(w):
    w = w.astype(jnp.float32)
    ms = []
    for s in (-1, 0, 1):
        m = jnp.zeros((R, R), jnp.float32)
        for kh in range(3):
            eye = jnp.eye(H, H, k=kh - 1, dtype=jnp.float32)
            m = m + jnp.kron(w[:, :, kh, s + 1], eye)
        ms.append(m)
    return ms


def _ode_kernel(x_ref, m1_ref, m2_ref, tw_ref, o_ref, stk_a, stk_b):
    nl = x_ref.shape[1]
    nl2 = nl // 2
    nb = nl2 // W
    for stk in (stk_a, stk_b):
        stk[:, :nb] = jnp.zeros((RS, nb), jnp.bfloat16)
        stk[:, nb + nl2:] = jnp.zeros((RS, nb), jnp.bfloat16)
        stk[R + 3:RA, nb:nb + nl2] = jnp.concatenate(
            [tw_ref[3:4], jnp.zeros((4, nl2), jnp.bfloat16)], axis=0)
        stk[RA:RA + R, nb:2 * nb] = jnp.zeros((R, nb), jnp.bfloat16)
        stk[RA + R:200, nl2:nb + nl2] = jnp.zeros((R, nb), jnp.bfloat16)
        stk[200:, nb:nb + nl2] = jnp.zeros((8, nl2), jnp.bfloat16)

    def conv3x3(slab16, m_ref, stk):
        stk[:R, nb:nb + nl2] = slab16
        stk[RA:RA + R, 2 * nb:2 * nb + nl2] = slab16
        stk[RA + R:200, :nl2] = slab16
        return jnp.dot(m_ref[...], stk[:, nb:nb + nl2],
                       preferred_element_type=jnp.float32)

    def odefunc(t, y16, stk):
        stk[R:R + 3, nb:nb + nl2] = t.astype(jnp.bfloat16) * tw_ref[:3]
        h = jnp.maximum(conv3x3(y16, m1_ref, stk).astype(jnp.bfloat16), 0)
        return conv3x3(h, m2_ref, stk)

    dt = jnp.float32(1.0 / N_STEPS)
    stks = (stk_a, stk_b)

    def rk4_step(i, ys):
        t = i.astype(jnp.float32) * dt
        k1 = [odefunc(t, ys[s].astype(jnp.bfloat16), stks[s])
              for s in (0, 1)]
        acc = [ys[s] + (dt / 6.0) * k1[s] for s in (0, 1)]
        k2 = [odefunc(t + 0.5 * dt,
                      (ys[s] + (0.5 * dt) * k1[s]).astype(jnp.bfloat16),
                      stks[s]) for s in (0, 1)]
        acc = [acc[s] + (dt / 3.0) * k2[s] for s in (0, 1)]
        k3 = [odefunc(t + 0.5 * dt,
                      (ys[s] + (0.5 * dt) * k2[s]).astype(jnp.bfloat16),
                      stks[s]) for s in (0, 1)]
        acc = [acc[s] + (dt / 3.0) * k3[s] for s in (0, 1)]
        k4 = [odefunc(t + dt,
                      (ys[s] + dt * k3[s]).astype(jnp.bfloat16),
                      stks[s]) for s in (0, 1)]
        return tuple(acc[s] + (dt / 6.0) * k4[s] for s in (0, 1))

    ya, yb = jax.lax.fori_loop(0, N_STEPS, rk4_step,
                               (x_ref[:, :nl2], x_ref[:, nl2:]))
    o_ref[:, :nl2] = ya
    o_ref[:, nl2:] = yb


def kernel(x, w1, b1, w2, b2):
    b = x.shape[0]
    nbs = NB if b % (2 * NB) == 0 else b // 2
    nl2 = nbs * W
    nl = 2 * nl2
    np_ = b // (2 * nbs)

    x = x.astype(jnp.float32)
    xp = (x.reshape(2 * np_, nbs, C, H, W)
           .transpose(2, 3, 0, 4, 1)
           .reshape(R, b * W))

    wv = jnp.arange(W)
    vw = jnp.stack([((wv + k - 1) >= 0) & ((wv + k - 1) < W)
                    for k in range(3)]).astype(jnp.float32)
    tw = jnp.concatenate(
        [jnp.repeat(vw, nbs, axis=1),
         jnp.ones((1, nl2), jnp.float32),
         jnp.zeros((4, nl2), jnp.float32)], axis=0).astype(jnp.bfloat16)

    hv = jnp.arange(H)
    vh = jnp.stack([((hv + k - 1) >= 0) & ((hv + k - 1) < H)
                    for k in range(3)]).astype(jnp.float32)
    m1t = jnp.einsum('ckl,kh->chl', w1[:, 0].astype(jnp.float32),
                     vh).reshape(R, 3)
    b1c = jnp.repeat(b1.astype(jnp.float32), H)[:, None]
    b2c = jnp.repeat(b2.astype(jnp.float32), H)[:, None]

    m1l, m1c, m1r = _band_weights(w1[:, 1:])
    m2l, m2c, m2r = _band_weights(w2)
    z4 = jnp.zeros((R, 4), jnp.float32)
    z3 = jnp.zeros((R, 3), jnp.float32)
    z8 = jnp.zeros((R, 8), jnp.float32)
    m1 = jnp.concatenate([m1c, m1t, b1c, z4, m1l, m1r, z8],
                         axis=1).astype(jnp.bfloat16)
    m2 = jnp.concatenate([m2c, z3, b2c, z4, m2l, m2r, z8],
                         axis=1).astype(jnp.bfloat16)

    scratch = pltpu.VMEM((RS, nl2 + 2 * nbs), jnp.bfloat16)
    out = pl.pallas_call(
        _ode_kernel,
        out_shape=jax.ShapeDtypeStruct((R, b * W), jnp.float32),
        grid=(np_,),
        in_specs=[
            pl.BlockSpec((R, nl), lambda p: (0, p)),
            pl.BlockSpec((R, RS), lambda p: (0, 0)),
            pl.BlockSpec((R, RS), lambda p: (0, 0)),
            pl.BlockSpec((8, nl2), lambda p: (0, 0)),
        ],
        out_specs=pl.BlockSpec((R, nl), lambda p: (0, p)),
        scratch_shapes=[scratch, scratch],
        compiler_params=pltpu.CompilerParams(
            dimension_semantics=("parallel",)),
    )(xp, m1, m2, tw)

    return (out.reshape(C, H, 2 * np_, W, nbs)
               .transpose(2, 4, 0, 1, 3)
               .reshape(b, C, H, W))


# speedup vs baseline: 1.0018x; 1.0018x over previous
import jax
import jax.numpy as jnp
from jax.experimental import pallas as pl
from jax.experimental.pallas import tpu as pltpu

C = 4
H = 16
W = 16
N_STEPS = 8
R = C * H
RA = R + 8
RS = 208
NB = 256


def _band_weights(w):
    w = w.astype(jnp.float32)
    ms = []
    for s in (-1, 0, 1):
        m = jnp.zeros((R, R), jnp.float32)
        for kh in range(3):
            eye = jnp.eye(H, H, k=kh - 1, dtype=jnp.float32)
            m = m + jnp.kron(w[:, :, kh, s + 1], eye)
        ms.append(m)
    return ms


def _ode_kernel(x_ref, m1_ref, m2_ref, tw_ref, o_ref, stk_a, stk_b):
    nl = x_ref.shape[1]
    nl2 = nl // 2
    nb = nl2 // W
    for stk in (stk_a, stk_b):
        stk[:, :nb] = jnp.zeros((RS, nb), jnp.bfloat16)
        stk[:, nb + nl2:] = jnp.zeros((RS, nb), jnp.bfloat16)
        stk[R + 3:RA, nb:nb + nl2] = jnp.concatenate(
            [tw_ref[3:4], jnp.zeros((4, nl2), jnp.bfloat16)], axis=0)
        stk[RA:RA + R, nb:2 * nb] = jnp.zeros((R, nb), jnp.bfloat16)
        stk[RA + R:200, nl2:nb + nl2] = jnp.zeros((R, nb), jnp.bfloat16)
        stk[200:, nb:nb + nl2] = jnp.zeros((8, nl2), jnp.bfloat16)

    def conv3x3(slab16, m_ref, stk):
        stk[:R, nb:nb + nl2] = slab16
        stk[RA:RA + R, 2 * nb:2 * nb + nl2] = slab16
        stk[RA + R:200, :nl2] = slab16
        return jnp.dot(m_ref[...], stk[:, nb:nb + nl2],
                       preferred_element_type=jnp.float32)

    def odefunc(t, y16, stk):
        stk[R:R + 3, nb:nb + nl2] = t.astype(jnp.bfloat16) * tw_ref[:3]
        h = jnp.maximum(conv3x3(y16, m1_ref, stk).astype(jnp.bfloat16), 0)
        return conv3x3(h, m2_ref, stk)

    dt = jnp.float32(1.0 / N_STEPS)
    stks = (stk_a, stk_b)

    def rk4_step(i, ys):
        t = i.astype(jnp.float32) * dt
        k1 = [odefunc(t, ys[s].astype(jnp.bfloat16), stks[s])
              for s in (0, 1)]
        acc = [ys[s] + (dt / 6.0) * k1[s] for s in (0, 1)]
        k2 = [odefunc(t + 0.5 * dt,
                      (ys[s] + (0.5 * dt) * k1[s]).astype(jnp.bfloat16),
                      stks[s]) for s in (0, 1)]
        acc = [acc[s] + (dt / 3.0) * k2[s] for s in (0, 1)]
        k3 = [odefunc(t + 0.5 * dt,
                      (ys[s] + (0.5 * dt) * k2[s]).astype(jnp.bfloat16),
                      stks[s]) for s in (0, 1)]
        acc = [acc[s] + (dt / 3.0) * k3[s] for s in (0, 1)]
        k4 = [odefunc(t + dt,
                      (ys[s] + dt * k3[s]).astype(jnp.bfloat16),
                      stks[s]) for s in (0, 1)]
        return tuple(acc[s] + (dt / 6.0) * k4[s] for s in (0, 1))

    ya, yb = jax.lax.fori_loop(0, N_STEPS, rk4_step,
                               (x_ref[:, :nl2], x_ref[:, nl2:]))
    o_ref[:, :nl2] = ya
    o_ref[:, nl2:] = yb


def kernel(x, w1, b1, w2, b2):
    b = x.shape[0]
    nbs = NB if b % (2 * NB) == 0 else b // 2
    nl2 = nbs * W
    nl = 2 * nl2
    np_ = b // (2 * nbs)

    x = x.astype(jnp.float32)
    xp = (x.reshape(2 * np_, nbs, C, H, W)
           .transpose(2, 3, 0, 4, 1)
           .reshape(R, b * W))

    wv = jnp.arange(W)
    vw = jnp.stack([((wv + k - 1) >= 0) & ((wv + k - 1) < W)
                    for k in range(3)]).astype(jnp.float32)
    tw = jnp.concatenate(
        [jnp.repeat(vw, nbs, axis=1),
         jnp.ones((1, nl2), jnp.float32),
         jnp.zeros((4, nl2), jnp.float32)], axis=0).astype(jnp.bfloat16)

    hv = jnp.arange(H)
    vh = jnp.stack([((hv + k - 1) >= 0) & ((hv + k - 1) < H)
                    for k in range(3)]).astype(jnp.float32)
    m1t = jnp.einsum('ckl,kh->chl', w1[:, 0].astype(jnp.float32),
                     vh).reshape(R, 3)
    b1c = jnp.repeat(b1.astype(jnp.float32), H)[:, None]
    b2c = jnp.repeat(b2.astype(jnp.float32), H)[:, None]

    m1l, m1c, m1r = _band_weights(w1[:, 1:])
    m2l, m2c, m2r = _band_weights(w2)
    z4 = jnp.zeros((R, 4), jnp.float32)
    z3 = jnp.zeros((R, 3), jnp.float32)
    z8 = jnp.zeros((R, 8), jnp.float32)
    m1 = jnp.concatenate([m1c, m1t, b1c, z4, m1l, m1r, z8],
                         axis=1).astype(jnp.bfloat16)
    m2 = jnp.concatenate([m2c, z3, b2c, z4, m2l, m2r, z8],
                         axis=1).astype(jnp.bfloat16)

    scratch = pltpu.VMEM((RS, nl2 + 2 * nbs), jnp.bfloat16)
    out = pl.pallas_call(
        _ode_kernel,
        out_shape=jax.ShapeDtypeStruct((R, b * W), jnp.float32),
        grid=(np_,),
        in_specs=[
            pl.BlockSpec((R, nl), lambda p: (0, p)),
            pl.BlockSpec((R, RS), lambda p: (0, 0)),
            pl.BlockSpec((R, RS), lambda p: (0, 0)),
            pl.BlockSpec((8, nl2), lambda p: (0, 0)),
        ],
        out_specs=pl.BlockSpec((R, nl), lambda p: (0, p)),
        scratch_shapes=[scratch, scratch],
        compiler_params=pltpu.CompilerParams(
            dimension_semantics=("parallel",)),
    )(xp, m1, m2, tw)

    return (out.reshape(C, H, 2 * np_, W, nbs)
               .transpose(2, 4, 0, 1, 3)
               .reshape(b, C, H, W))
